# baseline (speedup 1.0000x reference)
import functools

import jax
import jax.numpy as jnp
from jax import lax
from jax.experimental import pallas as pl
from jax.experimental.pallas import tpu as pltpu
from jax.experimental.pallas import tpu_sc as plsc

_NUM_CORES = 2
_NUM_SUBCORES = 16
_NUM_WORKERS = _NUM_CORES * _NUM_SUBCORES


def _make_sc_gather(num_rows: int, feat: int, chunk: int):
  assert num_rows % (_NUM_WORKERS * chunk) == 0
  rows_per_worker = num_rows // _NUM_WORKERS
  n_chunks = rows_per_worker // chunk
  mesh = plsc.VectorSubcoreMesh(core_axis_name="c", subcore_axis_name="s")

  @functools.partial(
      pl.kernel,
      mesh=mesh,
      compiler_params=pltpu.CompilerParams(use_tc_tiling_on_sc=False),
      out_type=jax.ShapeDtypeStruct((num_rows, feat), jnp.float32),
      scratch_types=[
          pltpu.VMEM((chunk,), jnp.int32),
          pltpu.VMEM((chunk, feat), jnp.float32),
          pltpu.SemaphoreType.DMA,
      ],
  )
  def gather_kernel(idx_hbm, table_hbm, out_hbm, idx_v, rows_v, sem):
    wid = lax.axis_index("s") * _NUM_CORES + lax.axis_index("c")
    wbase = wid * rows_per_worker

    @pl.loop(0, n_chunks)
    def _chunk_loop(g):
      base = wbase + g * chunk
      pltpu.sync_copy(idx_hbm.at[pl.ds(base, chunk)], idx_v)
      pltpu.async_copy(table_hbm.at[idx_v], rows_v, sem).wait()
      pltpu.sync_copy(rows_v, out_hbm.at[pl.ds(base, chunk)])

  return gather_kernel


def _mm_body(e_ref, w_ref, b_ref, o_ref):
  e = jnp.maximum(e_ref[...], 0.0)
  o_ref[...] = (
      jnp.dot(e, w_ref[...], preferred_element_type=jnp.float32) + b_ref[...]
  )


def _make_tc_matmul(num_rows: int, feat: int, ent: int, block_rows: int):
  assert num_rows % block_rows == 0
  grid = (num_rows // block_rows,)
  return pl.pallas_call(
      _mm_body,
      grid=grid,
      in_specs=[
          pl.BlockSpec((block_rows, feat), lambda i: (i, 0)),
          pl.BlockSpec((feat, ent), lambda i: (0, 0)),
          pl.BlockSpec((1, ent), lambda i: (0, 0)),
      ],
      out_specs=pl.BlockSpec((block_rows, ent), lambda i: (i, 0)),
      out_shape=jax.ShapeDtypeStruct((num_rows, ent), jnp.float32),
  )


def kernel(x, table, W, b):
  batch, fields = x.shape
  vocab, feat = table.shape
  ent = W.shape[1]
  num_rows = batch * fields

  xf = x.reshape(num_rows).astype(jnp.int32)
  e = _make_sc_gather(num_rows, feat, chunk=512)(xf, table)
  out = _make_tc_matmul(num_rows, feat, ent, block_rows=2048)(
      e, W, b.reshape(1, ent)
  )
  return out.reshape(batch, fields, ent)


# speedup vs baseline: 1.6752x; 1.6752x over previous
import functools

import jax
import jax.numpy as jnp
from jax import lax
from jax.experimental import pallas as pl
from jax.experimental.pallas import tpu as pltpu
from jax.experimental.pallas import tpu_sc as plsc

_NUM_CORES = 2
_NUM_SUBCORES = 16
_NUM_WORKERS = _NUM_CORES * _NUM_SUBCORES


def _make_sc_gather(num_rows: int, feat: int, chunk: int):
  assert num_rows % (_NUM_WORKERS * chunk) == 0
  rows_per_worker = num_rows // _NUM_WORKERS
  n_chunks = rows_per_worker // chunk
  mesh = plsc.VectorSubcoreMesh(core_axis_name="c", subcore_axis_name="s")

  @functools.partial(
      pl.kernel,
      mesh=mesh,
      compiler_params=pltpu.CompilerParams(use_tc_tiling_on_sc=False),
      out_type=jax.ShapeDtypeStruct((num_rows, feat), jnp.float32),
      scratch_types=[
          pltpu.VMEM((chunk,), jnp.int32),
          pltpu.VMEM((chunk, feat), jnp.float32),
          pltpu.SemaphoreType.DMA,
      ],
  )
  def gather_kernel(idx_hbm, table_hbm, out_hbm, idx_v, rows_v, sem):
    wid = lax.axis_index("s") * _NUM_CORES + lax.axis_index("c")
    wbase = wid * rows_per_worker

    @pl.loop(0, n_chunks)
    def _chunk_loop(g):
      base = wbase + g * chunk
      pltpu.sync_copy(idx_hbm.at[pl.ds(base, chunk)], idx_v)
      pltpu.async_copy(table_hbm.at[idx_v], rows_v, sem).wait()
      pltpu.sync_copy(rows_v, out_hbm.at[pl.ds(base, chunk)])

  return gather_kernel


def _mm_body(e_ref, w_ref, b_ref, o_ref):
  feat = w_ref.shape[0]
  w = w_ref[...]
  bias = b_ref[...]
  el = jnp.maximum(e_ref[:, :feat], 0.0)
  er = jnp.maximum(e_ref[:, feat:], 0.0)
  o_ref[0] = jnp.dot(el, w, preferred_element_type=jnp.float32) + bias
  o_ref[1] = jnp.dot(er, w, preferred_element_type=jnp.float32) + bias


def _make_tc_matmul(half_rows: int, feat: int, ent: int, block_rows: int):
  assert half_rows % block_rows == 0
  grid = (half_rows // block_rows,)
  return pl.pallas_call(
      _mm_body,
      grid=grid,
      in_specs=[
          pl.BlockSpec((block_rows, 2 * feat), lambda i: (i, 0)),
          pl.BlockSpec((feat, ent), lambda i: (0, 0)),
          pl.BlockSpec((1, ent), lambda i: (0, 0)),
      ],
      out_specs=pl.BlockSpec((2, block_rows, ent), lambda i: (0, i, 0)),
      out_shape=jax.ShapeDtypeStruct((2, half_rows, ent), jnp.float32),
  )


def kernel(x, table, W, b):
  batch, fields = x.shape
  vocab, feat = table.shape
  ent = W.shape[1]
  num_rows = batch * fields
  half = num_rows // 2

  xq = x.T.reshape(num_rows).astype(jnp.int32)
  xg = jnp.stack([xq[:half], xq[half:]], axis=1).reshape(num_rows)

  tlin = lax.optimization_barrier(table.reshape(vocab * feat))
  t2 = tlin.reshape(vocab, feat)

  e = _make_sc_gather(num_rows, feat, chunk=512)(xg, t2)
  e2 = e.reshape(half, 2 * feat)

  out3 = _make_tc_matmul(half, feat, ent, block_rows=2048)(
      e2, W, b.reshape(1, ent)
  )
  outq = out3.reshape(fields, batch, ent)
  return outq.transpose(1, 0, 2)


# speedup vs baseline: 1.9833x; 1.1839x over previous
import functools

import jax
import jax.numpy as jnp
from jax import lax
from jax.experimental import pallas as pl
from jax.experimental.pallas import tpu as pltpu
from jax.experimental.pallas import tpu_sc as plsc

_NUM_CORES = 2
_NUM_SUBCORES = 16
_NUM_WORKERS = _NUM_CORES * _NUM_SUBCORES

_PREP_COLS = 2048


def _make_tc_prep(vocab: int, feat: int):
  bc = _PREP_COLS
  h = bc // 2
  n_full = vocab // bc
  tail = vocab - n_full * bc
  ht = tail // 2
  grid = (n_full + 1,)

  def body(t_ref, o_ref):
    i = pl.program_id(0)
    v = t_ref[...]

    @pl.when(i < n_full)
    def _full():
      o_ref[...] = jnp.concatenate([v[:, :h].T, v[:, h:].T], axis=1)

    @pl.when(i == n_full)
    def _tail():
      o_ref[:ht] = jnp.concatenate([v[:, :ht].T, v[:, ht:tail].T], axis=1)

  return pl.pallas_call(
      body,
      grid=grid,
      in_specs=[pl.BlockSpec((feat, bc), lambda i: (0, i))],
      out_specs=pl.BlockSpec((h, 2 * feat), lambda i: (i, 0)),
      out_shape=jax.ShapeDtypeStruct((vocab // 2, 2 * feat), jnp.float32),
  )


def _remap_indices(r, vocab):
  bc = _PREP_COLS
  n_full = (vocab // bc) * bc
  base = (r // bc) * bc
  j = r - base
  hh = jnp.where(r < n_full, bc // 2, (vocab - n_full) // 2)
  return base + 2 * (j % hh) + j // hh


def _make_sc_gather(half_rows: int, feat: int, chunk: int):
  assert half_rows % (_NUM_WORKERS * chunk) == 0
  rows_per_worker = half_rows // _NUM_WORKERS
  n_chunks = rows_per_worker // chunk
  mesh = plsc.VectorSubcoreMesh(core_axis_name="c", subcore_axis_name="s")

  @functools.partial(
      pl.kernel,
      mesh=mesh,
      compiler_params=pltpu.CompilerParams(use_tc_tiling_on_sc=False),
      out_type=jax.ShapeDtypeStruct((half_rows, 2 * feat), jnp.float32),
      scratch_types=[
          pltpu.VMEM((chunk,), jnp.int32),
          pltpu.VMEM((chunk,), jnp.int32),
          pltpu.VMEM((chunk, feat), jnp.float32),
          pltpu.VMEM((chunk, feat), jnp.float32),
          pltpu.SemaphoreType.DMA,
          pltpu.SemaphoreType.DMA,
      ],
  )
  def gather_kernel(idxa_hbm, idxb_hbm, table_hbm, out_hbm,
                    idxa_v, idxb_v, rows_a, rows_b, sem_a, sem_b):
    wid = lax.axis_index("s") * _NUM_CORES + lax.axis_index("c")
    wbase = wid * rows_per_worker

    @pl.loop(0, n_chunks)
    def _chunk_loop(g):
      off = pl.multiple_of(wbase + g * chunk, chunk)
      pltpu.sync_copy(idxa_hbm.at[pl.ds(off, chunk)], idxa_v)
      pltpu.sync_copy(idxb_hbm.at[pl.ds(off, chunk)], idxb_v)
      cp_a = pltpu.async_copy(table_hbm.at[idxa_v], rows_a, sem_a)
      cp_b = pltpu.async_copy(table_hbm.at[idxb_v], rows_b, sem_b)
      cp_a.wait()
      pltpu.sync_copy(rows_a, out_hbm.at[pl.ds(off, chunk), pl.ds(0, feat)])
      cp_b.wait()
      pltpu.sync_copy(rows_b, out_hbm.at[pl.ds(off, chunk), pl.ds(feat, feat)])

  return gather_kernel


def _mm_body(e_ref, w_ref, b_ref, o_ref):
  feat = w_ref.shape[0]
  w = w_ref[...]
  bias = b_ref[...]
  el = jnp.maximum(e_ref[:, :feat], 0.0)
  er = jnp.maximum(e_ref[:, feat:], 0.0)
  o_ref[0] = jnp.dot(el, w, preferred_element_type=jnp.float32) + bias
  o_ref[1] = jnp.dot(er, w, preferred_element_type=jnp.float32) + bias


def _make_tc_matmul(half_rows: int, feat: int, ent: int, block_rows: int):
  assert half_rows % block_rows == 0
  grid = (half_rows // block_rows,)
  return pl.pallas_call(
      _mm_body,
      grid=grid,
      in_specs=[
          pl.BlockSpec((block_rows, 2 * feat), lambda i: (i, 0)),
          pl.BlockSpec((feat, ent), lambda i: (0, 0)),
          pl.BlockSpec((1, ent), lambda i: (0, 0)),
      ],
      out_specs=pl.BlockSpec((2, block_rows, ent), lambda i: (0, i, 0)),
      out_shape=jax.ShapeDtypeStruct((2, half_rows, ent), jnp.float32),
  )


def kernel(x, table, W, b):
  batch, fields = x.shape
  vocab, feat = table.shape
  ent = W.shape[1]
  num_rows = batch * fields
  half = num_rows // 2

  xq = x.T.reshape(num_rows).astype(jnp.int32)
  xr = _remap_indices(xq, vocab)
  xa = xr[:half]
  xb = xr[half:]

  tlin = _make_tc_prep(vocab, feat)(table.T)
  t2 = tlin.reshape(vocab, feat)

  e2 = _make_sc_gather(half, feat, chunk=256)(xa, xb, t2)

  out3 = _make_tc_matmul(half, feat, ent, block_rows=2048)(
      e2, W, b.reshape(1, ent)
  )
  outq = out3.reshape(fields, batch, ent)
  return outq.transpose(1, 0, 2)
